# baseline (device time: 13187 ns/iter reference)
import jax
import jax.numpy as jnp
from jax import lax
from jax.experimental import pallas as pl
from jax.experimental.pallas import tpu as pltpu

N_DEV = 4
N_HALF = 2
_DJ_ORDER = (2, 1, 3)


def _quantize(x):
    amax = jnp.maximum(jnp.max(jnp.abs(x)), 1e-20)
    scale = amax / 127.0
    q = jnp.round(x * (127.0 / amax)).astype(jnp.int8)
    return q, scale


def kernel(A, B):
    m, k = A.shape
    k2, n = B.shape
    chunk = m // N_DEV
    n2 = n // N_HALF

    def body(
        a_ref, b_ref, out_ref,
        stage1_ref, recv1_ref, sscale1_ref, rscale1_ref,
        stage2_ref, recv2_ref, sscale2_ref, rscale2_ref,
        s1_sems, r1_sems, s1s_sems, r1s_sems,
        s2_sems, r2_sems, s2s_sems, r2s_sems,
    ):
        my = lax.axis_index("i")

        barrier_sem = pltpu.get_barrier_semaphore()
        for dj in range(1, N_DEV):
            pl.semaphore_signal(
                barrier_sem, inc=1,
                device_id=((my + dj) % N_DEV,),
                device_id_type=pl.DeviceIdType.MESH,
            )

        b16 = b_ref[...].astype(jnp.bfloat16)

        def compute_half(h):
            for dj in _DJ_ORDER:
                peer = (my + dj) % N_DEV
                ck = jnp.dot(
                    a_ref[pl.ds(peer * chunk, chunk), :].astype(jnp.bfloat16),
                    b16[:, h * n2:(h + 1) * n2],
                    preferred_element_type=jnp.float32,
                )
                q, scale = _quantize(ck)
                stage1_ref[h, dj - 1, :, :] = q
                sscale1_ref[h, dj - 1, 0, :] = jnp.broadcast_to(scale, (n2,))

        def send_half(h):
            rdmas = {}
            for dj in _DJ_ORDER:
                peer = (my + dj) % N_DEV
                sc = pltpu.make_async_remote_copy(
                    src_ref=sscale1_ref.at[h, dj - 1],
                    dst_ref=rscale1_ref.at[h, dj - 1],
                    send_sem=s1s_sems.at[h, dj - 1],
                    recv_sem=r1s_sems.at[h, dj - 1],
                    device_id=(peer,),
                    device_id_type=pl.DeviceIdType.MESH,
                )
                sc.start()
                rdma = pltpu.make_async_remote_copy(
                    src_ref=stage1_ref.at[h, dj - 1],
                    dst_ref=recv1_ref.at[h, dj - 1],
                    send_sem=s1_sems.at[h, dj - 1],
                    recv_sem=r1_sems.at[h, dj - 1],
                    device_id=(peer,),
                    device_id_type=pl.DeviceIdType.MESH,
                )
                rdma.start()
                rdmas[dj] = (rdma, sc)
            return rdmas

        compute_half(0)
        pl.semaphore_wait(barrier_sem, N_DEV - 1)
        p1 = {0: send_half(0)}
        compute_half(1)
        p1[1] = send_half(1)

        my_ck = jnp.dot(
            a_ref[pl.ds(my * chunk, chunk), :].astype(jnp.bfloat16),
            b16,
            preferred_element_type=jnp.float32,
        )

        p2 = []
        for h in range(N_HALF):
            red = my_ck[:, h * n2:(h + 1) * n2]
            for dj in range(1, N_DEV):
                rdma, sc = p1[h][dj]
                rdma.wait_recv()
                sc.wait_recv()
                red = red + (
                    recv1_ref[h, dj - 1, :, :].astype(jnp.float32)
                    * rscale1_ref[h, dj - 1, 0, 0]
                )
            out_ref[pl.ds(my * chunk, chunk), pl.ds(h * n2, n2)] = (
                red.astype(jnp.bfloat16)
            )
            q, scale = _quantize(red)
            stage2_ref[h, :, :] = q
            sscale2_ref[h, 0, :] = jnp.broadcast_to(scale, (n2,))
            for dj in _DJ_ORDER:
                peer = (my + dj) % N_DEV
                sc = pltpu.make_async_remote_copy(
                    src_ref=sscale2_ref.at[h],
                    dst_ref=rscale2_ref.at[h, dj - 1],
                    send_sem=s2s_sems.at[h, dj - 1],
                    recv_sem=r2s_sems.at[h, dj - 1],
                    device_id=(peer,),
                    device_id_type=pl.DeviceIdType.MESH,
                )
                sc.start()
                rdma = pltpu.make_async_remote_copy(
                    src_ref=stage2_ref.at[h],
                    dst_ref=recv2_ref.at[h, dj - 1],
                    send_sem=s2_sems.at[h, dj - 1],
                    recv_sem=r2_sems.at[h, dj - 1],
                    device_id=(peer,),
                    device_id_type=pl.DeviceIdType.MESH,
                )
                rdma.start()
                p2.append((rdma, sc))

        for h in range(N_HALF):
            for s in range(N_DEV - 1):
                rdma, sc = p2[h * (N_DEV - 1) + s]
                rdma.wait_recv()
                sc.wait_recv()
                owner = (my - s - 1) % N_DEV
                out_ref[pl.ds(owner * chunk, chunk), pl.ds(h * n2, n2)] = (
                    recv2_ref[h, s, :, :].astype(jnp.float32)
                    * rscale2_ref[h, s, 0, 0]
                ).astype(jnp.bfloat16)

        for rdmas in p1.values():
            for rdma, sc in rdmas.values():
                rdma.wait_send()
                sc.wait_send()
        for rdma, sc in p2:
            rdma.wait_send()
            sc.wait_send()

    return pl.pallas_call(
        body,
        out_shape=jax.ShapeDtypeStruct((m, n), jnp.bfloat16),
        in_specs=[
            pl.BlockSpec(memory_space=pltpu.VMEM),
            pl.BlockSpec(memory_space=pltpu.VMEM),
        ],
        out_specs=pl.BlockSpec(memory_space=pltpu.VMEM),
        scratch_shapes=[
            pltpu.VMEM((N_HALF, N_DEV - 1, chunk, n2), jnp.int8),
            pltpu.VMEM((N_HALF, N_DEV - 1, chunk, n2), jnp.int8),
            pltpu.VMEM((N_HALF, N_DEV - 1, 1, n2), jnp.float32),
            pltpu.VMEM((N_HALF, N_DEV - 1, 1, n2), jnp.float32),
            pltpu.VMEM((N_HALF, chunk, n2), jnp.int8),
            pltpu.VMEM((N_HALF, N_DEV - 1, chunk, n2), jnp.int8),
            pltpu.VMEM((N_HALF, 1, n2), jnp.float32),
            pltpu.VMEM((N_HALF, N_DEV - 1, 1, n2), jnp.float32),
            pltpu.SemaphoreType.DMA((N_HALF, N_DEV - 1)),
            pltpu.SemaphoreType.DMA((N_HALF, N_DEV - 1)),
            pltpu.SemaphoreType.DMA((N_HALF, N_DEV - 1)),
            pltpu.SemaphoreType.DMA((N_HALF, N_DEV - 1)),
            pltpu.SemaphoreType.DMA((N_HALF, N_DEV - 1)),
            pltpu.SemaphoreType.DMA((N_HALF, N_DEV - 1)),
            pltpu.SemaphoreType.DMA((N_HALF, N_DEV - 1)),
            pltpu.SemaphoreType.DMA((N_HALF, N_DEV - 1)),
        ],
        compiler_params=pltpu.CompilerParams(collective_id=0),
    )(A, B)


# device time: 13102 ns/iter; 1.0065x vs baseline; 1.0065x over previous
import jax
import jax.numpy as jnp
from jax import lax
from jax.experimental import pallas as pl
from jax.experimental.pallas import tpu as pltpu

N_DEV = 4
N_HALF = 2
_DJ_ORDER = (2, 1, 3)


def _quantize(x):
    amax = jnp.maximum(jnp.max(jnp.abs(x)), 1e-20)
    scale = amax / 127.0
    q = jnp.round(x * (127.0 / amax)).astype(jnp.int8)
    return q, scale


def kernel(A, B):
    m, k = A.shape
    k2, n = B.shape
    chunk = m // N_DEV
    n2 = n // N_HALF

    def body(
        a_ref, b_ref, out_ref,
        partial_ref,
        stage1_ref, recv1_ref, sscale1_ref, rscale1_ref,
        stage2_ref, recv2_ref, sscale2_ref, rscale2_ref,
        s1_sems, r1_sems, s1s_sems, r1s_sems,
        s2_sems, r2_sems, s2s_sems, r2s_sems,
    ):
        my = lax.axis_index("i")

        barrier_sem = pltpu.get_barrier_semaphore()
        for dj in range(1, N_DEV):
            pl.semaphore_signal(
                barrier_sem, inc=1,
                device_id=((my + dj) % N_DEV,),
                device_id_type=pl.DeviceIdType.MESH,
            )

        p = jnp.dot(
            a_ref[...].astype(jnp.bfloat16),
            b_ref[...].astype(jnp.bfloat16),
            preferred_element_type=jnp.float32,
        )
        partial_ref[...] = p
        for j in range(N_DEV):
            for h in range(N_HALF):
                q, scale = _quantize(
                    p[j * chunk:(j + 1) * chunk, h * n2:(h + 1) * n2]
                )
                stage1_ref[h, j, :, :] = q
                sscale1_ref[h, j, 0, :] = jnp.broadcast_to(scale, (n2,))

        pl.semaphore_wait(barrier_sem, N_DEV - 1)

        p1 = {}
        for h in range(N_HALF):
            for dj in _DJ_ORDER:
                peer = (my + dj) % N_DEV
                sc = pltpu.make_async_remote_copy(
                    src_ref=sscale1_ref.at[h, peer],
                    dst_ref=rscale1_ref.at[h, dj - 1],
                    send_sem=s1s_sems.at[h, dj - 1],
                    recv_sem=r1s_sems.at[h, dj - 1],
                    device_id=(peer,),
                    device_id_type=pl.DeviceIdType.MESH,
                )
                sc.start()
                rdma = pltpu.make_async_remote_copy(
                    src_ref=stage1_ref.at[h, peer],
                    dst_ref=recv1_ref.at[h, dj - 1],
                    send_sem=s1_sems.at[h, dj - 1],
                    recv_sem=r1_sems.at[h, dj - 1],
                    device_id=(peer,),
                    device_id_type=pl.DeviceIdType.MESH,
                )
                rdma.start()
                p1[(h, dj)] = (rdma, sc)

        p2 = []
        for h in range(N_HALF):
            red = partial_ref[pl.ds(my * chunk, chunk), pl.ds(h * n2, n2)]
            for dj in range(1, N_DEV):
                rdma, sc = p1[(h, dj)]
                rdma.wait_recv()
                sc.wait_recv()
                red = red + (
                    recv1_ref[h, dj - 1, :, :].astype(jnp.float32)
                    * rscale1_ref[h, dj - 1, 0, 0]
                )
            out_ref[pl.ds(my * chunk, chunk), pl.ds(h * n2, n2)] = (
                red.astype(jnp.bfloat16)
            )
            q, scale = _quantize(red)
            stage2_ref[h, :, :] = q
            sscale2_ref[h, 0, :] = jnp.broadcast_to(scale, (n2,))
            for dj in _DJ_ORDER:
                peer = (my + dj) % N_DEV
                sc = pltpu.make_async_remote_copy(
                    src_ref=sscale2_ref.at[h],
                    dst_ref=rscale2_ref.at[h, dj - 1],
                    send_sem=s2s_sems.at[h, dj - 1],
                    recv_sem=r2s_sems.at[h, dj - 1],
                    device_id=(peer,),
                    device_id_type=pl.DeviceIdType.MESH,
                )
                sc.start()
                rdma = pltpu.make_async_remote_copy(
                    src_ref=stage2_ref.at[h],
                    dst_ref=recv2_ref.at[h, dj - 1],
                    send_sem=s2_sems.at[h, dj - 1],
                    recv_sem=r2_sems.at[h, dj - 1],
                    device_id=(peer,),
                    device_id_type=pl.DeviceIdType.MESH,
                )
                rdma.start()
                p2.append((rdma, sc))

        for h in range(N_HALF):
            for s in range(N_DEV - 1):
                rdma, sc = p2[h * (N_DEV - 1) + s]
                rdma.wait_recv()
                sc.wait_recv()
                owner = (my - s - 1) % N_DEV
                out_ref[pl.ds(owner * chunk, chunk), pl.ds(h * n2, n2)] = (
                    recv2_ref[h, s, :, :].astype(jnp.float32)
                    * rscale2_ref[h, s, 0, 0]
                ).astype(jnp.bfloat16)

        for rdma, sc in list(p1.values()) + p2:
            rdma.wait_send()
            sc.wait_send()

    return pl.pallas_call(
        body,
        out_shape=jax.ShapeDtypeStruct((m, n), jnp.bfloat16),
        in_specs=[
            pl.BlockSpec(memory_space=pltpu.VMEM),
            pl.BlockSpec(memory_space=pltpu.VMEM),
        ],
        out_specs=pl.BlockSpec(memory_space=pltpu.VMEM),
        scratch_shapes=[
            pltpu.VMEM((m, n), jnp.float32),
            pltpu.VMEM((N_HALF, N_DEV, chunk, n2), jnp.int8),
            pltpu.VMEM((N_HALF, N_DEV - 1, chunk, n2), jnp.int8),
            pltpu.VMEM((N_HALF, N_DEV, 1, n2), jnp.float32),
            pltpu.VMEM((N_HALF, N_DEV - 1, 1, n2), jnp.float32),
            pltpu.VMEM((N_HALF, chunk, n2), jnp.int8),
            pltpu.VMEM((N_HALF, N_DEV - 1, chunk, n2), jnp.int8),
            pltpu.VMEM((N_HALF, 1, n2), jnp.float32),
            pltpu.VMEM((N_HALF, N_DEV - 1, 1, n2), jnp.float32),
            pltpu.SemaphoreType.DMA((N_HALF, N_DEV - 1)),
            pltpu.SemaphoreType.DMA((N_HALF, N_DEV - 1)),
            pltpu.SemaphoreType.DMA((N_HALF, N_DEV - 1)),
            pltpu.SemaphoreType.DMA((N_HALF, N_DEV - 1)),
            pltpu.SemaphoreType.DMA((N_HALF, N_DEV - 1)),
            pltpu.SemaphoreType.DMA((N_HALF, N_DEV - 1)),
            pltpu.SemaphoreType.DMA((N_HALF, N_DEV - 1)),
            pltpu.SemaphoreType.DMA((N_HALF, N_DEV - 1)),
        ],
        compiler_params=pltpu.CompilerParams(collective_id=0),
    )(A, B)
